# Initial kernel scaffold; baseline (speedup 1.0000x reference)
import jax
import jax.numpy as jnp
from jax.experimental import pallas as pl


def kernel(x, edge_index, batch, W1, b1, W2, b2, Wlin, blin):
    raise NotImplementedError("write your pallas kernel here")


# speedup vs baseline: 7.8907x; 7.8907x over previous
import dataclasses
import functools

import jax
import jax.numpy as jnp
from jax import lax
from jax.experimental import pallas as pl
from jax.experimental.pallas import tpu as pltpu
from jax.experimental.pallas import tpu_sc as plsc

N = 10000
E = 320000
D = 128
OUT_D = 64
G = 16

NPAD = 10240
K = 80
CPT = 128
EPAD = 32 * CPT * K
NGRP = 16

_mesh = plsc.VectorSubcoreMesh(core_axis_name="c", subcore_axis_name="s")
_f32 = jnp.float32

_sc_params = pltpu.CompilerParams()
if "needs_layout_passes" in pltpu.CompilerParams.__dataclass_fields__:
    _sc_params = dataclasses.replace(_sc_params, needs_layout_passes=False)


@functools.partial(
    pl.kernel,
    out_type=jax.ShapeDtypeStruct((2, NPAD), _f32),
    mesh=_mesh,
    scratch_types=[
        pltpu.VMEM((8, K), jnp.int32),
        pltpu.VMEM((NPAD,), _f32),
        pltpu.VMEM((16, 128), _f32),
        pltpu.VMEM((128,), _f32),
        pltpu.VMEM_SHARED((16, NPAD), _f32),
    ],
    compiler_params=_sc_params,
)
def _sc_degree(dst_hbm, deg_hbm, didx, acc, buf, col, shared):
    c = lax.axis_index("c")
    s = lax.axis_index("s")
    tid = c * 16 + s
    ones16 = jnp.full((16,), 1.0, _f32)

    @pl.loop(0, NPAD // 16)
    def _(i):
        acc[pl.ds(i * 16, 16)] = jnp.zeros((16,), _f32)

    @pl.loop(0, NGRP)
    def _(g):
        pltpu.sync_copy(dst_hbm.at[tid, pl.ds(g * 8, 8), :], didx)
        for j in range(8):
            for q in range(K // 16):
                idx = didx[j, pl.ds(q * 16, 16)]
                plsc.addupdate_scatter(acc, [idx], ones16)

    pltpu.sync_copy(acc, shared.at[s])
    plsc.subcore_barrier()

    for p in range(5):
        base = s * 640 + p * 128
        pltpu.sync_copy(shared.at[:, pl.ds(base, 128)], buf)

        @pl.loop(0, 8)
        def _(g):
            sl = pl.ds(g * 16, 16)
            v = buf[0, sl]
            for r in range(1, 16):
                v = v + buf[r, sl]
            col[sl] = v

        pltpu.sync_copy(col, deg_hbm.at[c, pl.ds(base, 128)])


@functools.partial(
    pl.kernel,
    out_type=jax.ShapeDtypeStruct((2, NPAD, D), _f32),
    mesh=_mesh,
    scratch_types=[
        pltpu.VMEM((8, K), jnp.int32),
        pltpu.VMEM((8, K), jnp.int32),
        pltpu.VMEM((K, D), _f32),
        pltpu.VMEM_SHARED((NPAD, D), _f32),
    ],
    compiler_params=_sc_params,
)
def _sc_aggregate(u_hbm, src_hbm, dst_hbm, agg_hbm, sidx, didx, rows, acc):
    c = lax.axis_index("c")
    s = lax.axis_index("s")
    tid = c * 16 + s

    @pl.loop(0, K)
    def _(i):
        for q in range(D // 16):
            rows[i, pl.ds(q * 16, 16)] = jnp.zeros((16,), _f32)

    for k in range(8):
        pltpu.sync_copy(rows, acc.at[pl.ds(s * 640 + k * K, K), :])
    plsc.subcore_barrier()

    @pl.loop(0, NGRP)
    def _(g):
        pltpu.sync_copy(src_hbm.at[tid, pl.ds(g * 8, 8), :], sidx)
        pltpu.sync_copy(dst_hbm.at[tid, pl.ds(g * 8, 8), :], didx)
        for j in range(8):
            pltpu.sync_copy(u_hbm.at[sidx.at[j]], rows)
            pltpu.sync_copy(rows, acc.at[didx.at[j]], add=True)

    plsc.subcore_barrier()
    for k in range(8):
        base = s * 640 + k * K
        pltpu.sync_copy(acc.at[pl.ds(base, K), :], rows)
        pltpu.sync_copy(rows, agg_hbm.at[c, pl.ds(base, K), :])


_HI = lax.Precision.HIGHEST


def _dinv_from(deg):
    d = deg[0, :N, :] + deg[1, :N, :] + 1.0
    return lax.rsqrt(d)


def _tc_first_body(x_ref, w_ref, deg_ref, o_ref):
    dinv = _dinv_from(deg_ref[...])
    h = lax.dot_general(x_ref[...], w_ref[...], (((1,), (0,)), ((), ())),
                        precision=_HI)
    o_ref[...] = h * dinv


def _pre_activation(agg_ref, u_ref, deg_ref, b_ref):
    dinv = _dinv_from(deg_ref[...])
    pre = agg_ref[0, :N, :] + agg_ref[1, :N, :] + u_ref[...]
    return jnp.maximum(pre * dinv + b_ref[...], 0.0), dinv


def _tc_mid_body(agg_ref, u_ref, deg_ref, b_ref, w_ref, o_ref):
    h, dinv = _pre_activation(agg_ref, u_ref, deg_ref, b_ref)
    hw = lax.dot_general(h, w_ref[...], (((1,), (0,)), ((), ())),
                         precision=_HI)
    o_ref[...] = hw * dinv


def _tc_final_body(agg_ref, u_ref, deg_ref, b_ref, batch_ref, wlin_ref,
                   blin_ref, o_ref):
    h, _ = _pre_activation(agg_ref, u_ref, deg_ref, b_ref)
    seg = lax.broadcasted_iota(jnp.int32, (G, N), 0)
    mask = (seg == batch_ref[...]).astype(_f32)
    pooled = lax.dot_general(mask, h, (((1,), (0,)), ((), ())), precision=_HI)
    counts = jnp.sum(mask, axis=1, keepdims=True)
    gmean = pooled / jnp.maximum(counts, 1.0)
    out = lax.dot_general(gmean, wlin_ref[...], (((1,), (0,)), ((), ())),
                          precision=_HI) + blin_ref[...]
    nrm = jnp.sqrt(jnp.sum(out * out, axis=1, keepdims=True))
    o_ref[...] = out / jnp.maximum(nrm, 1e-12)


_tc_first = pl.pallas_call(
    _tc_first_body, out_shape=jax.ShapeDtypeStruct((N, D), _f32))
_tc_mid = pl.pallas_call(
    _tc_mid_body, out_shape=jax.ShapeDtypeStruct((N, D), _f32))
_tc_final = pl.pallas_call(
    _tc_final_body, out_shape=jax.ShapeDtypeStruct((G, OUT_D), _f32))


def kernel(x, edge_index, batch, W1, b1, W2, b2, Wlin, blin):
    npad_e = EPAD - E
    srcp = jnp.concatenate(
        [edge_index[0], jnp.zeros((npad_e,), jnp.int32)]).reshape(32, CPT, K)
    dstp = jnp.concatenate(
        [edge_index[1], jnp.full((npad_e,), NPAD - 1, jnp.int32)]
    ).reshape(32, CPT, K)
    b1r = b1.reshape(1, D)
    b2r = b2.reshape(1, D)
    blinr = blin.reshape(1, OUT_D)
    batch2d = batch.reshape(1, N)

    deg = _sc_degree(dstp).reshape(2, NPAD, 1)
    u1 = _tc_first(x, W1, deg)
    agg1 = _sc_aggregate(u1, srcp, dstp)
    u2 = _tc_mid(agg1, u1, deg, b1r, W2)
    agg2 = _sc_aggregate(u2, srcp, dstp)
    return _tc_final(agg2, u2, deg, b2r, batch2d, Wlin, blinr)


# speedup vs baseline: 7.9273x; 1.0046x over previous
import dataclasses
import functools

import jax
import jax.numpy as jnp
from jax import lax
from jax.experimental import pallas as pl
from jax.experimental.pallas import tpu as pltpu
from jax.experimental.pallas import tpu_sc as plsc

N = 10000
E = 320000
D = 128
OUT_D = 64
G = 16

NPAD = 10240
K = 80
CPT = 128
EPAD = 32 * CPT * K
NGRP = 16

_mesh = plsc.VectorSubcoreMesh(core_axis_name="c", subcore_axis_name="s")
_f32 = jnp.float32

_sc_params = pltpu.CompilerParams()
if "needs_layout_passes" in pltpu.CompilerParams.__dataclass_fields__:
    _sc_params = dataclasses.replace(_sc_params, needs_layout_passes=False)


@functools.partial(
    pl.kernel,
    out_type=jax.ShapeDtypeStruct((2, NPAD), _f32),
    mesh=_mesh,
    scratch_types=[
        pltpu.VMEM((8, K), jnp.int32),
        pltpu.VMEM((NPAD,), _f32),
        pltpu.VMEM((16, 128), _f32),
        pltpu.VMEM((128,), _f32),
        pltpu.VMEM_SHARED((16, NPAD), _f32),
    ],
    compiler_params=_sc_params,
)
def _sc_degree(dst_hbm, deg_hbm, didx, acc, buf, col, shared):
    c = lax.axis_index("c")
    s = lax.axis_index("s")
    tid = c * 16 + s
    ones16 = jnp.full((16,), 1.0, _f32)

    @pl.loop(0, NPAD // 16)
    def _(i):
        acc[pl.ds(i * 16, 16)] = jnp.zeros((16,), _f32)

    @pl.loop(0, NGRP)
    def _(g):
        pltpu.sync_copy(dst_hbm.at[tid, pl.ds(g * 8, 8), :], didx)
        for j in range(8):
            for q in range(K // 16):
                idx = didx[j, pl.ds(q * 16, 16)]
                plsc.addupdate_scatter(acc, [idx], ones16)

    pltpu.sync_copy(acc, shared.at[s])
    plsc.subcore_barrier()

    for p in range(5):
        base = s * 640 + p * 128
        pltpu.sync_copy(shared.at[:, pl.ds(base, 128)], buf)

        @pl.loop(0, 8)
        def _(g):
            sl = pl.ds(g * 16, 16)
            v = buf[0, sl]
            for r in range(1, 16):
                v = v + buf[r, sl]
            col[sl] = v

        pltpu.sync_copy(col, deg_hbm.at[c, pl.ds(base, 128)])


@functools.partial(
    pl.kernel,
    out_type=jax.ShapeDtypeStruct((2, NPAD, D), _f32),
    mesh=_mesh,
    scratch_types=[
        pltpu.VMEM((8, K), jnp.int32),
        pltpu.VMEM((8, K), jnp.int32),
        pltpu.VMEM((K, D), _f32),
        pltpu.VMEM_SHARED((NPAD, D), _f32),
    ],
    compiler_params=_sc_params,
)
def _sc_aggregate(u_hbm, src_hbm, dst_hbm, agg_hbm, sidx, didx, rows, acc):
    c = lax.axis_index("c")
    s = lax.axis_index("s")
    tid = c * 16 + s

    @pl.loop(0, K)
    def _(i):
        for q in range(D // 16):
            rows[i, pl.ds(q * 16, 16)] = jnp.zeros((16,), _f32)

    for k in range(8):
        pltpu.sync_copy(rows, acc.at[pl.ds(s * 640 + k * K, K), :])
    plsc.subcore_barrier()

    @pl.loop(0, NGRP)
    def _(g):
        pltpu.sync_copy(src_hbm.at[tid, pl.ds(g * 8, 8), :], sidx)
        pltpu.sync_copy(dst_hbm.at[tid, pl.ds(g * 8, 8), :], didx)
        for j in range(8):
            pltpu.sync_copy(u_hbm.at[sidx.at[j]], rows)
            pltpu.sync_copy(rows, acc.at[didx.at[j]], add=True)

    plsc.subcore_barrier()
    for k in range(8):
        base = s * 640 + k * K
        pltpu.sync_copy(acc.at[pl.ds(base, K), :], rows)
        pltpu.sync_copy(rows, agg_hbm.at[c, pl.ds(base, K), :])


_HI = lax.Precision.HIGHEST


def _dinv_from(deg):
    d = deg[0, :N, :] + deg[1, :N, :] + 1.0
    return lax.rsqrt(d)


def _tc_first_body(x_ref, w_ref, deg_ref, o_ref):
    dinv = _dinv_from(deg_ref[...])
    h = lax.dot_general(x_ref[...], w_ref[...], (((1,), (0,)), ((), ())),
                        precision=_HI)
    o_ref[...] = h * dinv


def _pre_activation(agg_ref, u_ref, deg_ref, b_ref):
    dinv = _dinv_from(deg_ref[...])
    pre = agg_ref[0, :N, :] + agg_ref[1, :N, :] + u_ref[...]
    return jnp.maximum(pre * dinv + b_ref[...], 0.0), dinv


def _tc_mid_body(agg_ref, u_ref, deg_ref, b_ref, w_ref, o_ref):
    h, dinv = _pre_activation(agg_ref, u_ref, deg_ref, b_ref)
    hw = lax.dot_general(h, w_ref[...], (((1,), (0,)), ((), ())),
                         precision=_HI)
    o_ref[...] = hw * dinv


def _tc_final_body(agg_ref, u_ref, deg_ref, b_ref, batch_ref, wlin_ref,
                   blin_ref, o_ref):
    h, _ = _pre_activation(agg_ref, u_ref, deg_ref, b_ref)
    seg = lax.broadcasted_iota(jnp.int32, (G, N), 0)
    mask = (seg == batch_ref[...]).astype(_f32)
    pooled = lax.dot_general(mask, h, (((1,), (0,)), ((), ())), precision=_HI)
    counts = jnp.sum(mask, axis=1, keepdims=True)
    gmean = pooled / jnp.maximum(counts, 1.0)
    out = lax.dot_general(gmean, wlin_ref[...], (((1,), (0,)), ((), ())),
                          precision=_HI) + blin_ref[...]
    nrm = jnp.sqrt(jnp.sum(out * out, axis=1, keepdims=True))
    o_ref[...] = out / jnp.maximum(nrm, 1e-12)


_tc_first = pl.pallas_call(
    _tc_first_body, out_shape=jax.ShapeDtypeStruct((N, D), _f32))
_tc_mid = pl.pallas_call(
    _tc_mid_body, out_shape=jax.ShapeDtypeStruct((N, D), _f32))
_tc_final = pl.pallas_call(
    _tc_final_body, out_shape=jax.ShapeDtypeStruct((G, OUT_D), _f32))


def kernel(x, edge_index, batch, W1, b1, W2, b2, Wlin, blin):
    npad_e = EPAD - E
    srcp = jnp.concatenate(
        [edge_index[0], jnp.zeros((npad_e,), jnp.int32)]).reshape(32, CPT, K)
    pad_dst = N + jnp.arange(npad_e, dtype=jnp.int32) % (NPAD - N)
    dstp = jnp.concatenate([edge_index[1], pad_dst]).reshape(32, CPT, K)
    b1r = b1.reshape(1, D)
    b2r = b2.reshape(1, D)
    blinr = blin.reshape(1, OUT_D)
    batch2d = batch.reshape(1, N)

    deg = _sc_degree(dstp).reshape(2, NPAD, 1)
    u1 = _tc_first(x, W1, deg)
    agg1 = _sc_aggregate(u1, srcp, dstp)
    u2 = _tc_mid(agg1, u1, deg, b1r, W2)
    agg2 = _sc_aggregate(u2, srcp, dstp)
    return _tc_final(agg2, u2, deg, b2r, batch2d, Wlin, blinr)


# speedup vs baseline: 8.5389x; 1.0771x over previous
import dataclasses
import functools

import jax
import jax.numpy as jnp
from jax import lax
from jax.experimental import pallas as pl
from jax.experimental.pallas import tpu as pltpu
from jax.experimental.pallas import tpu_sc as plsc

N = 10000
E = 320000
D = 128
OUT_D = 64
G = 16

NPAD = 10240
K = 80
CPT = 128
EPAD = 32 * CPT * K
NGRP = 16

_mesh = plsc.VectorSubcoreMesh(core_axis_name="c", subcore_axis_name="s")
_f32 = jnp.float32

_sc_params = pltpu.CompilerParams()
if "needs_layout_passes" in pltpu.CompilerParams.__dataclass_fields__:
    _sc_params = dataclasses.replace(_sc_params, needs_layout_passes=False)


@functools.partial(
    pl.kernel,
    out_type=jax.ShapeDtypeStruct((2, NPAD), _f32),
    mesh=_mesh,
    scratch_types=[
        pltpu.VMEM((8, K), jnp.int32),
        pltpu.VMEM((NPAD,), _f32),
        pltpu.VMEM((16, 128), _f32),
        pltpu.VMEM((128,), _f32),
        pltpu.VMEM_SHARED((16, NPAD), _f32),
    ],
    compiler_params=_sc_params,
)
def _sc_degree(dst_hbm, deg_hbm, didx, acc, buf, col, shared):
    c = lax.axis_index("c")
    s = lax.axis_index("s")
    tid = c * 16 + s
    ones16 = jnp.full((16,), 1.0, _f32)

    @pl.loop(0, NPAD // 16)
    def _(i):
        acc[pl.ds(i * 16, 16)] = jnp.zeros((16,), _f32)

    @pl.loop(0, NGRP)
    def _(g):
        pltpu.sync_copy(dst_hbm.at[tid, pl.ds(g * 8, 8), :], didx)
        for j in range(8):
            for q in range(K // 16):
                idx = didx[j, pl.ds(q * 16, 16)]
                plsc.addupdate_scatter(acc, [idx], ones16)

    pltpu.sync_copy(acc, shared.at[s])
    plsc.subcore_barrier()

    for p in range(5):
        base = s * 640 + p * 128
        pltpu.sync_copy(shared.at[:, pl.ds(base, 128)], buf)

        @pl.loop(0, 8)
        def _(g):
            sl = pl.ds(g * 16, 16)
            v = buf[0, sl]
            for r in range(1, 16):
                v = v + buf[r, sl]
            col[sl] = v

        pltpu.sync_copy(col, deg_hbm.at[c, pl.ds(base, 128)])


@functools.partial(
    pl.kernel,
    out_type=jax.ShapeDtypeStruct((2, NPAD, D), _f32),
    mesh=_mesh,
    scratch_types=[
        pltpu.VMEM((8, K), jnp.int32),
        pltpu.VMEM((8, K), jnp.int32),
        pltpu.VMEM((K, D), _f32),
        pltpu.VMEM((K, D), _f32),
        pltpu.SemaphoreType.DMA,
        pltpu.SemaphoreType.DMA,
        pltpu.SemaphoreType.DMA,
        pltpu.SemaphoreType.DMA,
        pltpu.VMEM_SHARED((NPAD, D), _f32),
    ],
    compiler_params=_sc_params,
)
def _sc_aggregate(u_hbm, src_hbm, dst_hbm, agg_hbm, sidx, didx, rows_a, rows_b,
                  gsem_a, gsem_b, ssem_a, ssem_b, acc):
    c = lax.axis_index("c")
    s = lax.axis_index("s")
    tid = c * 16 + s

    @pl.loop(0, K)
    def _(i):
        for q in range(D // 16):
            rows_a[i, pl.ds(q * 16, 16)] = jnp.zeros((16,), _f32)

    for k in range(8):
        pltpu.sync_copy(rows_a, acc.at[pl.ds(s * 640 + k * K, K), :])
    plsc.subcore_barrier()

    @pl.loop(0, NGRP)
    def _(g):
        pltpu.sync_copy(src_hbm.at[tid, pl.ds(g * 8, 8), :], sidx)
        pltpu.sync_copy(dst_hbm.at[tid, pl.ds(g * 8, 8), :], didx)
        bufs = ((rows_a, gsem_a, ssem_a), (rows_b, gsem_b, ssem_b))
        gat = [None, None]
        sca = [None, None]
        gat[0] = pltpu.async_copy(u_hbm.at[sidx.at[0]], rows_a, gsem_a)
        for j in range(8):
            rows, gsem, ssem = bufs[j % 2]
            gat[j % 2].wait()
            if j < 7:
                nrows, ngsem, _ = bufs[(j + 1) % 2]
                if j >= 1:
                    sca[(j + 1) % 2].wait()
                gat[(j + 1) % 2] = pltpu.async_copy(
                    u_hbm.at[sidx.at[j + 1]], nrows, ngsem)
            sca[j % 2] = pltpu.async_copy(rows, acc.at[didx.at[j]], ssem,
                                          add=True)
        sca[0].wait()
        sca[1].wait()

    plsc.subcore_barrier()
    for k in range(4):
        base = s * 640 + 2 * k * K
        cp_a = pltpu.async_copy(acc.at[pl.ds(base, K), :], rows_a, gsem_a)
        cp_b = pltpu.async_copy(acc.at[pl.ds(base + K, K), :], rows_b, gsem_b)
        cp_a.wait()
        out_a = pltpu.async_copy(rows_a, agg_hbm.at[c, pl.ds(base, K), :],
                                 ssem_a)
        cp_b.wait()
        out_b = pltpu.async_copy(rows_b, agg_hbm.at[c, pl.ds(base + K, K), :],
                                 ssem_b)
        out_a.wait()
        out_b.wait()


_HI = lax.Precision.HIGHEST


def _dinv_from(deg):
    d = deg[0, :N, :] + deg[1, :N, :] + 1.0
    return lax.rsqrt(d)


def _tc_first_body(x_ref, w_ref, deg_ref, o_ref):
    dinv = _dinv_from(deg_ref[...])
    h = lax.dot_general(x_ref[...], w_ref[...], (((1,), (0,)), ((), ())),
                        precision=_HI)
    o_ref[...] = h * dinv


def _pre_activation(agg_ref, u_ref, deg_ref, b_ref):
    dinv = _dinv_from(deg_ref[...])
    pre = agg_ref[0, :N, :] + agg_ref[1, :N, :] + u_ref[...]
    return jnp.maximum(pre * dinv + b_ref[...], 0.0), dinv


def _tc_mid_body(agg_ref, u_ref, deg_ref, b_ref, w_ref, o_ref):
    h, dinv = _pre_activation(agg_ref, u_ref, deg_ref, b_ref)
    hw = lax.dot_general(h, w_ref[...], (((1,), (0,)), ((), ())),
                         precision=_HI)
    o_ref[...] = hw * dinv


def _tc_final_body(agg_ref, u_ref, deg_ref, b_ref, batch_ref, wlin_ref,
                   blin_ref, o_ref):
    h, _ = _pre_activation(agg_ref, u_ref, deg_ref, b_ref)
    seg = lax.broadcasted_iota(jnp.int32, (G, N), 0)
    mask = (seg == batch_ref[...]).astype(_f32)
    pooled = lax.dot_general(mask, h, (((1,), (0,)), ((), ())), precision=_HI)
    counts = jnp.sum(mask, axis=1, keepdims=True)
    gmean = pooled / jnp.maximum(counts, 1.0)
    out = lax.dot_general(gmean, wlin_ref[...], (((1,), (0,)), ((), ())),
                          precision=_HI) + blin_ref[...]
    nrm = jnp.sqrt(jnp.sum(out * out, axis=1, keepdims=True))
    o_ref[...] = out / jnp.maximum(nrm, 1e-12)


_tc_first = pl.pallas_call(
    _tc_first_body, out_shape=jax.ShapeDtypeStruct((N, D), _f32))
_tc_mid = pl.pallas_call(
    _tc_mid_body, out_shape=jax.ShapeDtypeStruct((N, D), _f32))
_tc_final = pl.pallas_call(
    _tc_final_body, out_shape=jax.ShapeDtypeStruct((G, OUT_D), _f32))


def kernel(x, edge_index, batch, W1, b1, W2, b2, Wlin, blin):
    npad_e = EPAD - E
    srcp = jnp.concatenate(
        [edge_index[0], jnp.zeros((npad_e,), jnp.int32)]).reshape(32, CPT, K)
    pad_dst = N + jnp.arange(npad_e, dtype=jnp.int32) % (NPAD - N)
    dstp = jnp.concatenate([edge_index[1], pad_dst]).reshape(32, CPT, K)
    b1r = b1.reshape(1, D)
    b2r = b2.reshape(1, D)
    blinr = blin.reshape(1, OUT_D)
    batch2d = batch.reshape(1, N)

    deg = _sc_degree(dstp).reshape(2, NPAD, 1)
    u1 = _tc_first(x, W1, deg)
    agg1 = _sc_aggregate(u1, srcp, dstp)
    u2 = _tc_mid(agg1, u1, deg, b1r, W2)
    agg2 = _sc_aggregate(u2, srcp, dstp)
    return _tc_final(agg2, u2, deg, b2r, batch2d, Wlin, blinr)


# speedup vs baseline: 9.3134x; 1.0907x over previous
import dataclasses
import functools

import jax
import jax.numpy as jnp
from jax import lax
from jax.experimental import pallas as pl
from jax.experimental.pallas import tpu as pltpu
from jax.experimental.pallas import tpu_sc as plsc

N = 10000
E = 320000
D = 128
OUT_D = 64
G = 16

NPAD = 10240
EPAD = 327680
BLK = 128
NG = EPAD // (8 * BLK)
W0 = 10
W1 = 10
DK = 80
DNG = EPAD // (8 * DK)

_mesh = plsc.VectorSubcoreMesh(core_axis_name="c", subcore_axis_name="s")
_f32 = jnp.float32

_sc_params = pltpu.CompilerParams()
if "needs_layout_passes" in pltpu.CompilerParams.__dataclass_fields__:
    _sc_params = dataclasses.replace(_sc_params, needs_layout_passes=False)


@functools.partial(
    pl.kernel,
    out_type=jax.ShapeDtypeStruct((2, NPAD), _f32),
    mesh=_mesh,
    scratch_types=[
        pltpu.VMEM((8, DK), jnp.int32),
        pltpu.VMEM((NPAD,), _f32),
        pltpu.VMEM((16, 128), _f32),
        pltpu.VMEM((128,), _f32),
        pltpu.VMEM_SHARED((16, NPAD), _f32),
    ],
    compiler_params=_sc_params,
)
def _sc_degree(dst_hbm, deg_hbm, didx, acc, buf, col, shared):
    c = lax.axis_index("c")
    s = lax.axis_index("s")
    tid = s * 2 + c
    ones16 = jnp.full((16,), 1.0, _f32)

    @pl.loop(0, NPAD // 16)
    def _(i):
        acc[pl.ds(i * 16, 16)] = jnp.zeros((16,), _f32)

    @pl.loop(0, DNG // 32)
    def _(g):
        pltpu.sync_copy(dst_hbm.at[tid * (DNG // 32) + g], didx)
        for j in range(8):
            for q in range(DK // 16):
                idx = didx[j, pl.ds(q * 16, 16)]
                plsc.addupdate_scatter(acc, [idx], ones16)

    pltpu.sync_copy(acc, shared.at[s])
    plsc.subcore_barrier()

    for p in range(5):
        base = s * 640 + p * 128
        pltpu.sync_copy(shared.at[:, pl.ds(base, 128)], buf)

        @pl.loop(0, 8)
        def _(g):
            sl = pl.ds(g * 16, 16)
            v = buf[0, sl]
            for r in range(1, 16):
                v = v + buf[r, sl]
            col[sl] = v

        pltpu.sync_copy(col, deg_hbm.at[c, pl.ds(base, 128)])


@functools.partial(
    pl.kernel,
    out_type=jax.ShapeDtypeStruct((2, NPAD, D), _f32),
    mesh=_mesh,
    scratch_types=[
        pltpu.VMEM((8, BLK), jnp.int32),
        pltpu.VMEM((8, BLK), jnp.int32),
        pltpu.VMEM((BLK, D), _f32),
        pltpu.VMEM((BLK, D), _f32),
        pltpu.SemaphoreType.DMA,
        pltpu.SemaphoreType.DMA,
        pltpu.SemaphoreType.DMA,
        pltpu.SemaphoreType.DMA,
        pltpu.VMEM_SHARED((NPAD, D), _f32),
    ],
    compiler_params=_sc_params,
)
def _sc_aggregate(u_hbm, src_hbm, dst_hbm, agg_hbm, sidx, didx, rows_a, rows_b,
                  gsem_a, gsem_b, ssem_a, ssem_b, acc):
    c = lax.axis_index("c")
    s = lax.axis_index("s")

    @pl.loop(0, BLK)
    def _(i):
        for q in range(D // 16):
            rows_a[i, pl.ds(q * 16, 16)] = jnp.zeros((16,), _f32)

    for k in range(640 // BLK):
        pltpu.sync_copy(rows_a, acc.at[pl.ds(s * 640 + k * BLK, BLK), :])
    plsc.subcore_barrier()

    ng = jnp.where(c == 0, W0, W1)
    base = jnp.where(c == 0, s * W0, 16 * W0 + s * W1)

    @pl.loop(0, max(W0, W1))
    def _(g):
        @pl.when(g < ng)
        def _():
            gg = base + g
            pltpu.sync_copy(src_hbm.at[gg], sidx)
            pltpu.sync_copy(dst_hbm.at[gg], didx)
            bufs = ((rows_a, gsem_a, ssem_a), (rows_b, gsem_b, ssem_b))
            gat = [None, None]
            sca = [None, None]
            gat[0] = pltpu.async_copy(u_hbm.at[sidx.at[0]], rows_a, gsem_a)
            for j in range(8):
                rows, gsem, ssem = bufs[j % 2]
                gat[j % 2].wait()
                if j < 7:
                    nrows, ngsem, _ = bufs[(j + 1) % 2]
                    if j >= 1:
                        sca[(j + 1) % 2].wait()
                    gat[(j + 1) % 2] = pltpu.async_copy(
                        u_hbm.at[sidx.at[j + 1]], nrows, ngsem)
                sca[j % 2] = pltpu.async_copy(rows, acc.at[didx.at[j]], ssem,
                                              add=True)
            sca[0].wait()
            sca[1].wait()

    plsc.subcore_barrier()
    for k in range(0, 640 // BLK, 2):
        base = s * 640 + k * BLK
        cp_a = pltpu.async_copy(acc.at[pl.ds(base, BLK), :], rows_a, gsem_a)
        cp_b = pltpu.async_copy(acc.at[pl.ds(base + BLK, BLK), :], rows_b,
                                gsem_b)
        cp_a.wait()
        out_a = pltpu.async_copy(rows_a, agg_hbm.at[c, pl.ds(base, BLK), :],
                                 ssem_a)
        cp_b.wait()
        out_b = pltpu.async_copy(rows_b,
                                 agg_hbm.at[c, pl.ds(base + BLK, BLK), :],
                                 ssem_b)
        out_a.wait()
        out_b.wait()


_HI = lax.Precision.HIGHEST


def _dinv_from(deg):
    d = deg[0, :N, :] + deg[1, :N, :] + 1.0
    return lax.rsqrt(d)


def _tc_first_body(x_ref, w_ref, deg_ref, o_ref):
    dinv = _dinv_from(deg_ref[...])
    h = lax.dot_general(x_ref[...], w_ref[...], (((1,), (0,)), ((), ())),
                        precision=_HI)
    o_ref[...] = h * dinv


def _pre_activation(agg_ref, u_ref, deg_ref, b_ref):
    dinv = _dinv_from(deg_ref[...])
    pre = agg_ref[0, :N, :] + agg_ref[1, :N, :] + u_ref[...]
    return jnp.maximum(pre * dinv + b_ref[...], 0.0), dinv


def _tc_mid_body(agg_ref, u_ref, deg_ref, b_ref, w_ref, o_ref):
    h, dinv = _pre_activation(agg_ref, u_ref, deg_ref, b_ref)
    hw = lax.dot_general(h, w_ref[...], (((1,), (0,)), ((), ())),
                         precision=_HI)
    o_ref[...] = hw * dinv


def _tc_final_body(agg_ref, u_ref, deg_ref, b_ref, batch_ref, wlin_ref,
                   blin_ref, o_ref):
    h, _ = _pre_activation(agg_ref, u_ref, deg_ref, b_ref)
    seg = lax.broadcasted_iota(jnp.int32, (G, N), 0)
    mask = (seg == batch_ref[...]).astype(_f32)
    pooled = lax.dot_general(mask, h, (((1,), (0,)), ((), ())), precision=_HI)
    counts = jnp.sum(mask, axis=1, keepdims=True)
    gmean = pooled / jnp.maximum(counts, 1.0)
    out = lax.dot_general(gmean, wlin_ref[...], (((1,), (0,)), ((), ())),
                          precision=_HI) + blin_ref[...]
    nrm = jnp.sqrt(jnp.sum(out * out, axis=1, keepdims=True))
    o_ref[...] = out / jnp.maximum(nrm, 1e-12)


_tc_first = pl.pallas_call(
    _tc_first_body, out_shape=jax.ShapeDtypeStruct((N, D), _f32))
_tc_mid = pl.pallas_call(
    _tc_mid_body, out_shape=jax.ShapeDtypeStruct((N, D), _f32))
_tc_final = pl.pallas_call(
    _tc_final_body, out_shape=jax.ShapeDtypeStruct((G, OUT_D), _f32))


def kernel(x, edge_index, batch, W1, b1, W2, b2, Wlin, blin):
    npad_e = EPAD - E
    src_flat = jnp.concatenate(
        [edge_index[0], jnp.zeros((npad_e,), jnp.int32)])
    pad_dst = N + jnp.arange(npad_e, dtype=jnp.int32) % (NPAD - N)
    dst_flat = jnp.concatenate([edge_index[1], pad_dst])
    srcp = src_flat.reshape(NG, 8, BLK)
    dstp = dst_flat.reshape(NG, 8, BLK)
    dstp8 = dst_flat.reshape(DNG, 8, DK)
    b1r = b1.reshape(1, D)
    b2r = b2.reshape(1, D)
    blinr = blin.reshape(1, OUT_D)
    batch2d = batch.reshape(1, N)

    deg = _sc_degree(dstp8).reshape(2, NPAD, 1)
    u1 = _tc_first(x, W1, deg)
    agg1 = _sc_aggregate(u1, srcp, dstp)
    u2 = _tc_mid(agg1, u1, deg, b1r, W2)
    agg2 = _sc_aggregate(u2, srcp, dstp)
    return _tc_final(agg2, u2, deg, b2r, batch2d, Wlin, blinr)


# speedup vs baseline: 10.0056x; 1.0743x over previous
import dataclasses
import functools

import jax
import jax.numpy as jnp
from jax import lax
from jax.experimental import pallas as pl
from jax.experimental.pallas import tpu as pltpu
from jax.experimental.pallas import tpu_sc as plsc

N = 10000
E = 320000
D = 128
OUT_D = 64
G = 16

NPAD = 10240
EPAD = 327680
BLK = 128
NG = EPAD // (8 * BLK)
W0 = 13
W1 = 7
DK = 80
DNG = EPAD // (8 * DK)

_mesh = plsc.VectorSubcoreMesh(core_axis_name="c", subcore_axis_name="s")
_f32 = jnp.float32

_sc_params = pltpu.CompilerParams()
if "needs_layout_passes" in pltpu.CompilerParams.__dataclass_fields__:
    _sc_params = dataclasses.replace(_sc_params, needs_layout_passes=False)


@functools.partial(
    pl.kernel,
    out_type=jax.ShapeDtypeStruct((2, NPAD), _f32),
    mesh=_mesh,
    scratch_types=[
        pltpu.VMEM((8, DK), jnp.int32),
        pltpu.VMEM((NPAD,), _f32),
        pltpu.VMEM((16, 128), _f32),
        pltpu.VMEM((128,), _f32),
        pltpu.VMEM_SHARED((16, NPAD), _f32),
    ],
    compiler_params=_sc_params,
)
def _sc_degree(dst_hbm, deg_hbm, didx, acc, buf, col, shared):
    c = lax.axis_index("c")
    s = lax.axis_index("s")
    tid = s * 2 + c
    ones16 = jnp.full((16,), 1.0, _f32)

    @pl.loop(0, NPAD // 16)
    def _(i):
        acc[pl.ds(i * 16, 16)] = jnp.zeros((16,), _f32)

    @pl.loop(0, DNG // 32)
    def _(g):
        pltpu.sync_copy(dst_hbm.at[tid * (DNG // 32) + g], didx)
        for j in range(8):
            for q in range(DK // 16):
                idx = didx[j, pl.ds(q * 16, 16)]
                plsc.addupdate_scatter(acc, [idx], ones16)

    pltpu.sync_copy(acc, shared.at[s])
    plsc.subcore_barrier()

    for p in range(5):
        base = s * 640 + p * 128
        pltpu.sync_copy(shared.at[:, pl.ds(base, 128)], buf)

        @pl.loop(0, 8)
        def _(g):
            sl = pl.ds(g * 16, 16)
            v = buf[0, sl]
            for r in range(1, 16):
                v = v + buf[r, sl]
            col[sl] = v

        pltpu.sync_copy(col, deg_hbm.at[c, pl.ds(base, 128)])


@functools.partial(
    pl.kernel,
    out_type=jax.ShapeDtypeStruct((2, NPAD, D), _f32),
    mesh=_mesh,
    scratch_types=[
        pltpu.VMEM((8, BLK), jnp.int32),
        pltpu.VMEM((8, BLK), jnp.int32),
        pltpu.VMEM((BLK, D), _f32),
        pltpu.VMEM((BLK, D), _f32),
        pltpu.SemaphoreType.DMA,
        pltpu.SemaphoreType.DMA,
        pltpu.SemaphoreType.DMA,
        pltpu.SemaphoreType.DMA,
        pltpu.VMEM_SHARED((NPAD, D), _f32),
    ],
    compiler_params=_sc_params,
)
def _sc_aggregate(u_hbm, src_hbm, dst_hbm, agg_hbm, sidx, didx, rows_a, rows_b,
                  gsem_a, gsem_b, ssem_a, ssem_b, acc):
    c = lax.axis_index("c")
    s = lax.axis_index("s")

    @pl.loop(0, BLK)
    def _(i):
        for q in range(D // 16):
            rows_a[i, pl.ds(q * 16, 16)] = jnp.zeros((16,), _f32)

    for k in range(640 // BLK):
        pltpu.sync_copy(rows_a, acc.at[pl.ds(s * 640 + k * BLK, BLK), :])
    plsc.subcore_barrier()

    ng = jnp.where(c == 0, W0, W1)
    base = jnp.where(c == 0, s * W0, 16 * W0 + s * W1)

    @pl.loop(0, max(W0, W1))
    def _(g):
        @pl.when(g < ng)
        def _():
            gg = base + g
            pltpu.sync_copy(src_hbm.at[gg], sidx)
            pltpu.sync_copy(dst_hbm.at[gg], didx)
            bufs = ((rows_a, gsem_a, ssem_a), (rows_b, gsem_b, ssem_b))
            gat = [None, None]
            sca = [None, None]
            gat[0] = pltpu.async_copy(u_hbm.at[sidx.at[0]], rows_a, gsem_a)
            for j in range(8):
                rows, gsem, ssem = bufs[j % 2]
                gat[j % 2].wait()
                if j < 7:
                    nrows, ngsem, _ = bufs[(j + 1) % 2]
                    if j >= 1:
                        sca[(j + 1) % 2].wait()
                    gat[(j + 1) % 2] = pltpu.async_copy(
                        u_hbm.at[sidx.at[j + 1]], nrows, ngsem)
                sca[j % 2] = pltpu.async_copy(rows, acc.at[didx.at[j]], ssem,
                                              add=True)
            sca[0].wait()
            sca[1].wait()

    plsc.subcore_barrier()
    for k in range(0, 640 // BLK, 2):
        base = s * 640 + k * BLK
        cp_a = pltpu.async_copy(acc.at[pl.ds(base, BLK), :], rows_a, gsem_a)
        cp_b = pltpu.async_copy(acc.at[pl.ds(base + BLK, BLK), :], rows_b,
                                gsem_b)
        cp_a.wait()
        out_a = pltpu.async_copy(rows_a, agg_hbm.at[c, pl.ds(base, BLK), :],
                                 ssem_a)
        cp_b.wait()
        out_b = pltpu.async_copy(rows_b,
                                 agg_hbm.at[c, pl.ds(base + BLK, BLK), :],
                                 ssem_b)
        out_a.wait()
        out_b.wait()


_HI = lax.Precision.HIGHEST


def _dinv_from(deg):
    d = deg[0, :N, :] + deg[1, :N, :] + 1.0
    return lax.rsqrt(d)


def _tc_first_body(x_ref, w_ref, deg_ref, o_ref):
    dinv = _dinv_from(deg_ref[...])
    h = lax.dot_general(x_ref[...], w_ref[...], (((1,), (0,)), ((), ())),
                        precision=_HI)
    o_ref[...] = h * dinv


def _pre_activation(agg_ref, u_ref, deg_ref, b_ref):
    dinv = _dinv_from(deg_ref[...])
    pre = agg_ref[0, :N, :] + agg_ref[1, :N, :] + u_ref[...]
    return jnp.maximum(pre * dinv + b_ref[...], 0.0), dinv


def _tc_mid_body(agg_ref, u_ref, deg_ref, b_ref, w_ref, o_ref):
    h, dinv = _pre_activation(agg_ref, u_ref, deg_ref, b_ref)
    hw = lax.dot_general(h, w_ref[...], (((1,), (0,)), ((), ())),
                         precision=_HI)
    o_ref[...] = hw * dinv


def _tc_final_body(agg_ref, u_ref, deg_ref, b_ref, batch_ref, wlin_ref,
                   blin_ref, o_ref):
    h, _ = _pre_activation(agg_ref, u_ref, deg_ref, b_ref)
    seg = lax.broadcasted_iota(jnp.int32, (G, N), 0)
    mask = (seg == batch_ref[...]).astype(_f32)
    pooled = lax.dot_general(mask, h, (((1,), (0,)), ((), ())), precision=_HI)
    counts = jnp.sum(mask, axis=1, keepdims=True)
    gmean = pooled / jnp.maximum(counts, 1.0)
    out = lax.dot_general(gmean, wlin_ref[...], (((1,), (0,)), ((), ())),
                          precision=_HI) + blin_ref[...]
    nrm = jnp.sqrt(jnp.sum(out * out, axis=1, keepdims=True))
    o_ref[...] = out / jnp.maximum(nrm, 1e-12)


_tc_first = pl.pallas_call(
    _tc_first_body, out_shape=jax.ShapeDtypeStruct((N, D), _f32))
_tc_mid = pl.pallas_call(
    _tc_mid_body, out_shape=jax.ShapeDtypeStruct((N, D), _f32))
_tc_final = pl.pallas_call(
    _tc_final_body, out_shape=jax.ShapeDtypeStruct((G, OUT_D), _f32))


def kernel(x, edge_index, batch, W1, b1, W2, b2, Wlin, blin):
    npad_e = EPAD - E
    src_flat = jnp.concatenate(
        [edge_index[0], jnp.zeros((npad_e,), jnp.int32)])
    pad_dst = N + jnp.arange(npad_e, dtype=jnp.int32) % (NPAD - N)
    dst_flat = jnp.concatenate([edge_index[1], pad_dst])
    srcp = src_flat.reshape(NG, 8, BLK)
    dstp = dst_flat.reshape(NG, 8, BLK)
    dstp8 = dst_flat.reshape(DNG, 8, DK)
    b1r = b1.reshape(1, D)
    b2r = b2.reshape(1, D)
    blinr = blin.reshape(1, OUT_D)
    batch2d = batch.reshape(1, N)

    deg = _sc_degree(dstp8).reshape(2, NPAD, 1)
    u1 = _tc_first(x, W1, deg)
    agg1 = _sc_aggregate(u1, srcp, dstp)
    u2 = _tc_mid(agg1, u1, deg, b1r, W2)
    agg2 = _sc_aggregate(u2, srcp, dstp)
    return _tc_final(agg2, u2, deg, b2r, batch2d, Wlin, blinr)
